# baseline (speedup 1.0000x reference)
import jax
import jax.numpy as jnp
from jax import lax
from jax.experimental import pallas as pl
from jax.experimental.pallas import tpu as pltpu
from jax.experimental.pallas import tpu_sc as plsc
import functools

_NC = 2
_NS = 16
_NW = _NC * _NS
_CHUNK = 16
_NSLOT = 3


def _sc_body(nchunks, hid, x_hbm, pos_hbm, out_hbm, bufx, bufp,
             sx0, sx1, sx2, sp0, sp1, sp2, so0, so1, so2):
    cid = lax.axis_index("c")
    sid = lax.axis_index("s")
    wid = sid * _NC + cid
    rows_per_w = nchunks * _CHUNK
    base = wid * rows_per_w
    pbase = lax.rem(base, pos_hbm.shape[0])
    per_row = hid // 16
    sx = (sx0, sx1, sx2)
    sp = (sp0, sp1, sp2)
    so = (so0, so1, so2)
    x_cp = [None] * nchunks
    p_cp = [None] * nchunks
    out_cp = [None] * nchunks

    def load(m):
        t = m % _NSLOT
        x_cp[m] = pltpu.async_copy(
            x_hbm.at[pl.ds(base + m * _CHUNK, _CHUNK)], bufx.at[t], sx[t])
        p_cp[m] = pltpu.async_copy(
            pos_hbm.at[pl.ds(pbase + m * _CHUNK, _CHUNK)], bufp.at[t], sp[t])

    load(0)
    if nchunks > 1:
        load(1)
    for c in range(nchunks):
        s = c % _NSLOT
        m = c + 2
        if m < nchunks:
            if c >= 1:
                out_cp[c - 1].wait()
            load(m)
        x_cp[c].wait()
        p_cp[c].wait()

        @pl.loop(0, _CHUNK)
        def _(r):
            @pl.loop(0, per_row, unroll=8)
            def _(j):
                off = pl.multiple_of(j * 16, 16)
                plsc.addupdate(bufx.at[s, r, pl.ds(off, 16)],
                               bufp[s, r, pl.ds(off, 16)])

        out_cp[c] = pltpu.async_copy(
            bufx.at[s], out_hbm.at[pl.ds(base + c * _CHUNK, _CHUNK)], so[s])
    for c in range(max(0, nchunks - 3), nchunks):
        out_cp[c].wait()


def _sc_add(x2, pos_embedding):
    rows, hid = x2.shape
    rows_per_w = rows // _NW
    nchunks = rows_per_w // _CHUNK
    mesh = plsc.VectorSubcoreMesh(core_axis_name="c", subcore_axis_name="s")
    return pl.kernel(
        functools.partial(_sc_body, nchunks, hid),
        out_type=jax.ShapeDtypeStruct((rows, hid), x2.dtype),
        mesh=mesh,
        scratch_types=[
            pltpu.VMEM((_NSLOT, _CHUNK, hid), jnp.float32),
            pltpu.VMEM((_NSLOT, _CHUNK, hid), jnp.float32),
            pltpu.SemaphoreType.DMA,
            pltpu.SemaphoreType.DMA,
            pltpu.SemaphoreType.DMA,
            pltpu.SemaphoreType.DMA,
            pltpu.SemaphoreType.DMA,
            pltpu.SemaphoreType.DMA,
            pltpu.SemaphoreType.DMA,
            pltpu.SemaphoreType.DMA,
            pltpu.SemaphoreType.DMA,
        ],
    )(x2, pos_embedding)


def kernel(x, pos_embedding):
    batch, seq, hid = x.shape
    x2 = x.reshape(batch * seq, hid)
    out2 = _sc_add(x2, pos_embedding)
    return out2.reshape(batch, seq, hid)


# speedup vs baseline: 1.8892x; 1.8892x over previous
import jax
import jax.numpy as jnp
from jax import lax
from jax.experimental import pallas as pl
from jax.experimental.pallas import tpu as pltpu
from jax.experimental.pallas import tpu_sc as plsc
import functools

_NC = 2
_NS = 16
_NW = _NC * _NS
_CHUNK = 32


def _sc_body(nchunks, hid, x_hbm, pos_hbm, out_hbm,
             bufx, bufp, sx0, sx1, sp, so0, so1):
    cid = lax.axis_index("c")
    sid = lax.axis_index("s")
    wid = sid * _NC + cid
    rows_per_w = nchunks * _CHUNK
    base = wid * rows_per_w
    pbase = lax.rem(base, pos_hbm.shape[0])
    per_row = hid // 16
    nvec = _CHUNK * per_row
    sx = (sx0, sx1)
    so = (so0, so1)
    x_cp = [None] * nchunks
    p_cp = [None] * nchunks
    out_cp = [None] * nchunks
    x_cp[0] = pltpu.async_copy(x_hbm.at[pl.ds(base, _CHUNK)], bufx.at[0], sx[0])
    p_cp[0] = pltpu.async_copy(pos_hbm.at[pl.ds(pbase, _CHUNK)], bufp, sp)
    for c in range(nchunks):
        s = c & 1
        if c + 1 < nchunks:
            if c >= 1:
                out_cp[c - 1].wait()
            x_cp[c + 1] = pltpu.async_copy(
                x_hbm.at[pl.ds(base + (c + 1) * _CHUNK, _CHUNK)],
                bufx.at[1 - s], sx[1 - s])
        x_cp[c].wait()
        p_cp[c].wait()

        @plsc.parallel_loop(0, nvec, unroll=8)
        def _(i):
            r = i // per_row
            off = pl.multiple_of((i % per_row) * 16, 16)
            plsc.addupdate(bufx.at[s, r, pl.ds(off, 16)], bufp[r, pl.ds(off, 16)])

        if c + 1 < nchunks:
            p_cp[c + 1] = pltpu.async_copy(
                pos_hbm.at[pl.ds(pbase + (c + 1) * _CHUNK, _CHUNK)], bufp, sp)
        out_cp[c] = pltpu.async_copy(
            bufx.at[s], out_hbm.at[pl.ds(base + c * _CHUNK, _CHUNK)], so[s])
    out_cp[nchunks - 1].wait()
    if nchunks >= 2:
        out_cp[nchunks - 2].wait()


def _sc_add(x2, pos_embedding):
    rows, hid = x2.shape
    rows_per_w = rows // _NW
    nchunks = rows_per_w // _CHUNK
    mesh = plsc.VectorSubcoreMesh(core_axis_name="c", subcore_axis_name="s")
    return pl.kernel(
        functools.partial(_sc_body, nchunks, hid),
        out_type=jax.ShapeDtypeStruct((rows, hid), x2.dtype),
        mesh=mesh,
        scratch_types=[
            pltpu.VMEM((2, _CHUNK, hid), jnp.float32),
            pltpu.VMEM((_CHUNK, hid), jnp.float32),
            pltpu.SemaphoreType.DMA,
            pltpu.SemaphoreType.DMA,
            pltpu.SemaphoreType.DMA,
            pltpu.SemaphoreType.DMA,
            pltpu.SemaphoreType.DMA,
        ],
    )(x2, pos_embedding)


def kernel(x, pos_embedding):
    batch, seq, hid = x.shape
    x2 = x.reshape(batch * seq, hid)
    out2 = _sc_add(x2, pos_embedding)
    return out2.reshape(batch, seq, hid)


_BLK_S = 2048


def _add_body(x_ref, pos_ref, o_ref):
    o_ref[...] = x_ref[...] + pos_ref[...][None, :, :]


def _tc_kernel(x, pos_embedding):
    batch, seq, hid = x.shape
    grid = (seq // _BLK_S, batch)
    return pl.pallas_call(
        _add_body,
        grid=grid,
        in_specs=[
            pl.BlockSpec((1, _BLK_S, hid), lambda s, b: (b, s, 0)),
            pl.BlockSpec((_BLK_S, hid), lambda s, b: (s, 0)),
        ],
        out_specs=pl.BlockSpec((1, _BLK_S, hid), lambda s, b: (b, s, 0)),
        out_shape=jax.ShapeDtypeStruct(x.shape, x.dtype),
        compiler_params=pltpu.CompilerParams(
            dimension_semantics=("arbitrary", "arbitrary"),
        ),
    )(x, pos_embedding)


# speedup vs baseline: 2.2733x; 1.2034x over previous
import jax
import jax.numpy as jnp
from jax import lax
from jax.experimental import pallas as pl
from jax.experimental.pallas import tpu as pltpu
from jax.experimental.pallas import tpu_sc as plsc
import functools

_NC = 2
_NS = 16
_NW = _NC * _NS
_XCH = 16
_PCH = 32


def _sc_body(batch, seq, hid, x_hbm, pos_hbm, out_hbm,
             bufx, bufp, sx0, sx1, sp0, sp1, so0, so1):
    cid = lax.axis_index("c")
    sid = lax.axis_index("s")
    wid = sid * _NC + cid
    span = seq // _NW
    pstart = wid * span
    npos = span // _PCH
    xc_per_group = (_PCH // _XCH) * batch
    nx = npos * xc_per_group
    per_row = hid // 16
    nvec = _XCH * per_row
    sx = (sx0, sx1)
    sp = (sp0, sp1)
    so = (so0, so1)

    def xrow(g):
        p, r = divmod(g, xc_per_group)
        h, b = divmod(r, batch)
        return b * seq + pstart + p * _PCH + h * _XCH, h

    x_cp = [None] * nx
    p_cp = [None] * npos
    out_cp = [None] * nx

    def load_x(g):
        row, _ = xrow(g)
        x_cp[g] = pltpu.async_copy(
            x_hbm.at[pl.ds(row, _XCH)], bufx.at[g & 1], sx[g & 1])

    def load_p(p):
        p_cp[p] = pltpu.async_copy(
            pos_hbm.at[pl.ds(pstart + p * _PCH, _PCH)], bufp.at[p & 1],
            sp[p & 1])

    load_p(0)
    if npos > 1:
        load_p(1)
    load_x(0)
    for g in range(nx):
        s = g & 1
        p, r = divmod(g, xc_per_group)
        if g + 1 < nx:
            if g >= 1:
                out_cp[g - 1].wait()
            load_x(g + 1)
        if r == 0:
            p_cp[p].wait()
        _, h = xrow(g)

        x_cp[g].wait()

        @plsc.parallel_loop(0, nvec, unroll=8)
        def _(i):
            rr = i // per_row
            off = pl.multiple_of((i % per_row) * 16, 16)
            plsc.addupdate(bufx.at[s, rr, pl.ds(off, 16)],
                           bufp[p & 1, h * _XCH + rr, pl.ds(off, 16)])

        if r == xc_per_group - 1 and p + 2 < npos:
            load_p(p + 2)
        row, _ = xrow(g)
        out_cp[g] = pltpu.async_copy(
            bufx.at[s], out_hbm.at[pl.ds(row, _XCH)], so[s])
    out_cp[nx - 1].wait()
    if nx >= 2:
        out_cp[nx - 2].wait()


def _sc_add(x, pos_embedding):
    batch, seq, hid = x.shape
    x2 = x.reshape(batch * seq, hid)
    mesh = plsc.VectorSubcoreMesh(core_axis_name="c", subcore_axis_name="s")
    out2 = pl.kernel(
        functools.partial(_sc_body, batch, seq, hid),
        out_type=jax.ShapeDtypeStruct((batch * seq, hid), x.dtype),
        mesh=mesh,
        scratch_types=[
            pltpu.VMEM((2, _XCH, hid), jnp.float32),
            pltpu.VMEM((2, _PCH, hid), jnp.float32),
            pltpu.SemaphoreType.DMA,
            pltpu.SemaphoreType.DMA,
            pltpu.SemaphoreType.DMA,
            pltpu.SemaphoreType.DMA,
            pltpu.SemaphoreType.DMA,
            pltpu.SemaphoreType.DMA,
        ],
    )(x2, pos_embedding)
    return out2.reshape(batch, seq, hid)


def kernel(x, pos_embedding):
    return _sc_add(x, pos_embedding)


# speedup vs baseline: 2.3761x; 1.0452x over previous
import jax
import jax.numpy as jnp
from jax import lax
from jax.experimental import pallas as pl
from jax.experimental.pallas import tpu as pltpu
from jax.experimental.pallas import tpu_sc as plsc
import functools

_NC = 2
_NS = 16
_NW = _NC * _NS
_XCH = 16
_PCH = 32


def _sc_body(batch, seq, hid, x_hbm, pos_hbm, out_hbm,
             bufx, bufp, sx0, sx1, sx2, sp0, sp1, so0, so1, so2):
    cid = lax.axis_index("c")
    sid = lax.axis_index("s")
    wid = sid * _NC + cid
    span = seq // _NW
    pstart = wid * span
    npos = span // _PCH
    xc_per_group = (_PCH // _XCH) * batch
    nx = npos * xc_per_group
    per_row = hid // 16
    nvec = _XCH * per_row
    sx = (sx0, sx1, sx2)
    sp = (sp0, sp1)
    so = (so0, so1, so2)

    def xrow(g):
        p, r = divmod(g, xc_per_group)
        h, b = divmod(r, batch)
        return b * seq + pstart + p * _PCH + h * _XCH, h

    x_cp = [None] * nx
    p_cp = [None] * npos
    out_cp = [None] * nx

    def load_x(g):
        row, _ = xrow(g)
        x_cp[g] = pltpu.async_copy(
            x_hbm.at[pl.ds(row, _XCH)], bufx.at[g % 3], sx[g % 3])

    def load_p(p):
        p_cp[p] = pltpu.async_copy(
            pos_hbm.at[pl.ds(pstart + p * _PCH, _PCH)], bufp.at[p & 1],
            sp[p & 1])

    load_p(0)
    if npos > 1:
        load_p(1)
    load_x(0)
    if nx > 1:
        load_x(1)
    for g in range(nx):
        s = g % 3
        p, r = divmod(g, xc_per_group)
        if g + 2 < nx:
            if g >= 1:
                out_cp[g - 1].wait()
            load_x(g + 2)
        if r == 0:
            p_cp[p].wait()
        _, h = xrow(g)

        x_cp[g].wait()

        @plsc.parallel_loop(0, nvec, unroll=8)
        def _(i):
            rr = i // per_row
            off = pl.multiple_of((i % per_row) * 16, 16)
            plsc.addupdate(bufx.at[s, rr, pl.ds(off, 16)],
                           bufp[p & 1, h * _XCH + rr, pl.ds(off, 16)])

        if r == xc_per_group - 1 and p + 2 < npos:
            load_p(p + 2)
        row, _ = xrow(g)
        out_cp[g] = pltpu.async_copy(
            bufx.at[s], out_hbm.at[pl.ds(row, _XCH)], so[s])
    for g in range(max(0, nx - 3), nx):
        out_cp[g].wait()


def _sc_add(x, pos_embedding):
    batch, seq, hid = x.shape
    x2 = x.reshape(batch * seq, hid)
    mesh = plsc.VectorSubcoreMesh(core_axis_name="c", subcore_axis_name="s")
    out2 = pl.kernel(
        functools.partial(_sc_body, batch, seq, hid),
        out_type=jax.ShapeDtypeStruct((batch * seq, hid), x.dtype),
        mesh=mesh,
        scratch_types=[
            pltpu.VMEM((3, _XCH, hid), jnp.float32),
            pltpu.VMEM((2, _PCH, hid), jnp.float32),
            pltpu.SemaphoreType.DMA,
            pltpu.SemaphoreType.DMA,
            pltpu.SemaphoreType.DMA,
            pltpu.SemaphoreType.DMA,
            pltpu.SemaphoreType.DMA,
            pltpu.SemaphoreType.DMA,
            pltpu.SemaphoreType.DMA,
            pltpu.SemaphoreType.DMA,
        ],
    )(x2, pos_embedding)
    return out2.reshape(batch, seq, hid)


def kernel(x, pos_embedding):
    return _sc_add(x, pos_embedding)


# speedup vs baseline: 2.4015x; 1.0107x over previous
import jax
import jax.numpy as jnp
from jax import lax
from jax.experimental import pallas as pl
from jax.experimental.pallas import tpu as pltpu
from jax.experimental.pallas import tpu_sc as plsc
import functools

_NC = 2
_NS = 16
_NW = _NC * _NS
_XCH = 16
_PCH = 16


def _sc_body(batch, seq, hid, x_hbm, pos_hbm, out_hbm,
             bufx, bufp, sx0, sx1, sx2, sx3, sp0, sp1, so0, so1, so2, so3):
    cid = lax.axis_index("c")
    sid = lax.axis_index("s")
    wid = sid * _NC + cid
    span = seq // _NW
    pstart = wid * span
    npos = span // _PCH
    xc_per_group = (_PCH // _XCH) * batch
    nx = npos * xc_per_group
    per_row = hid // 16
    nvec = _XCH * per_row
    sx = (sx0, sx1, sx2, sx3)
    sp = (sp0, sp1)
    so = (so0, so1, so2, so3)

    def xrow(g):
        p, r = divmod(g, xc_per_group)
        h, b = divmod(r, batch)
        return b * seq + pstart + p * _PCH + h * _XCH, h

    x_cp = [None] * nx
    p_cp = [None] * npos
    out_cp = [None] * nx

    def load_x(g):
        row, _ = xrow(g)
        x_cp[g] = pltpu.async_copy(
            x_hbm.at[pl.ds(row, _XCH)], bufx.at[g % 4], sx[g % 4])

    def load_p(p):
        p_cp[p] = pltpu.async_copy(
            pos_hbm.at[pl.ds(pstart + p * _PCH, _PCH)], bufp.at[p & 1],
            sp[p & 1])

    load_p(0)
    if npos > 1:
        load_p(1)
    for g0 in range(min(3, nx)):
        load_x(g0)
    for g in range(nx):
        s = g % 4
        p, r = divmod(g, xc_per_group)
        if g + 3 < nx:
            if g >= 1:
                out_cp[g - 1].wait()
            load_x(g + 3)
        if r == 0:
            p_cp[p].wait()
        _, h = xrow(g)

        x_cp[g].wait()

        @plsc.parallel_loop(0, nvec, unroll=8)
        def _(i):
            rr = i // per_row
            off = pl.multiple_of((i % per_row) * 16, 16)
            plsc.addupdate(bufx.at[s, rr, pl.ds(off, 16)],
                           bufp[p & 1, h * _XCH + rr, pl.ds(off, 16)])

        if r == xc_per_group - 1 and p + 2 < npos:
            load_p(p + 2)
        row, _ = xrow(g)
        out_cp[g] = pltpu.async_copy(
            bufx.at[s], out_hbm.at[pl.ds(row, _XCH)], so[s])
    for g in range(max(0, nx - 4), nx):
        out_cp[g].wait()


def _sc_add(x, pos_embedding):
    batch, seq, hid = x.shape
    x2 = x.reshape(batch * seq, hid)
    mesh = plsc.VectorSubcoreMesh(core_axis_name="c", subcore_axis_name="s")
    out2 = pl.kernel(
        functools.partial(_sc_body, batch, seq, hid),
        out_type=jax.ShapeDtypeStruct((batch * seq, hid), x.dtype),
        mesh=mesh,
        scratch_types=[
            pltpu.VMEM((4, _XCH, hid), jnp.float32),
            pltpu.VMEM((2, _PCH, hid), jnp.float32),
            pltpu.SemaphoreType.DMA,
            pltpu.SemaphoreType.DMA,
            pltpu.SemaphoreType.DMA,
            pltpu.SemaphoreType.DMA,
            pltpu.SemaphoreType.DMA,
            pltpu.SemaphoreType.DMA,
            pltpu.SemaphoreType.DMA,
            pltpu.SemaphoreType.DMA,
            pltpu.SemaphoreType.DMA,
            pltpu.SemaphoreType.DMA,
        ],
    )(x2, pos_embedding)
    return out2.reshape(batch, seq, hid)


def kernel(x, pos_embedding):
    return _sc_add(x, pos_embedding)
